# Initial kernel scaffold; baseline (speedup 1.0000x reference)
import jax
import jax.numpy as jnp
from jax.experimental import pallas as pl


def kernel(weights):
    raise NotImplementedError("write your pallas kernel here")


# speedup vs baseline: 21.1166x; 21.1166x over previous
import jax
import jax.numpy as jnp
from jax import lax
from jax.experimental import pallas as pl
from jax.experimental.pallas import tpu as pltpu

_K = 256
_TAU = 1.0
_N = 1_000_000
_LANES = 128
_ROWS = 7816
_NPAD = _ROWS * _LANES


def _floor_avg(lo, hi):
    return (lo >> 1) + (hi >> 1) + (lo & hi & 1)


def _topk_mask_body(scores_ref, mask_ref, keys_ref):
    s = scores_ref[...]
    i = lax.bitcast_convert_type(s, jnp.int32)
    keys = jnp.where(i >= 0, i, i ^ jnp.int32(0x7FFFFFFF))
    keys_ref[...] = keys

    def search_body(_, carry):
        lo, hi = carry
        mid = _floor_avg(lo, hi)
        cnt = jnp.sum((keys_ref[...] > mid).astype(jnp.int32))
        take = cnt >= _K
        return jnp.where(take, mid, lo), jnp.where(take, hi, mid)

    lo0 = jnp.int32(-(2**31))
    hi0 = jnp.int32(2**31 - 1)
    _, v_k = lax.fori_loop(0, 32, search_body, (lo0, hi0))

    keysv = keys_ref[...]
    above = keysv > v_k
    n_above = jnp.sum(above.astype(jnp.int32))
    tie = keysv == v_k
    n_tie = jnp.sum(tie.astype(jnp.int32))
    need = _K - n_above

    lin = (lax.broadcasted_iota(jnp.int32, (_ROWS, _LANES), 0) * _LANES
           + lax.broadcasted_iota(jnp.int32, (_ROWS, _LANES), 1))

    def tie_cut_search(_):
        def body(_, carry):
            lo, hi = carry
            mid = _floor_avg(lo, hi)
            cnt = jnp.sum((tie & (lin <= mid)).astype(jnp.int32))
            take = cnt >= need
            return jnp.where(take, lo, mid), jnp.where(take, mid, hi)

        _, cut = lax.fori_loop(0, 20, body, (jnp.int32(-1), jnp.int32(_NPAD - 1)))
        return cut

    cut = lax.cond(n_tie == need, lambda _: jnp.int32(_NPAD), tie_cut_search,
                   operand=None)
    mask_ref[...] = (above | (tie & (lin <= cut))).astype(jnp.float32)


def _topk_mask(scores_pad2d):
    return pl.pallas_call(
        _topk_mask_body,
        out_shape=jax.ShapeDtypeStruct((_ROWS, _LANES), jnp.float32),
        scratch_shapes=[pltpu.VMEM((_ROWS, _LANES), jnp.int32)],
    )(scores_pad2d)


def kernel(weights):
    u = jax.random.uniform(jax.random.key(42), weights.shape, dtype=weights.dtype)
    u = jnp.clip(u, 1e-20, None)
    gumbel = -jnp.log(-jnp.log(u))
    scores = (jnp.log(jnp.clip(weights, 1e-20, None)) + gumbel) / _TAU
    spad = jnp.pad(scores, (0, _NPAD - _N),
                   constant_values=-jnp.inf).reshape(_ROWS, _LANES)
    mask2d = _topk_mask(spad)
    return mask2d.reshape(-1)[:_N]
